# Initial kernel scaffold; baseline (speedup 1.0000x reference)
import jax
import jax.numpy as jnp
from jax.experimental import pallas as pl


def kernel(x, router_w, w1, w2, bias):
    raise NotImplementedError("write your pallas kernel here")


# speedup vs baseline: 1.3261x; 1.3261x over previous
import jax
import jax.numpy as jnp
from jax.experimental import pallas as pl
from jax.experimental.pallas import tpu as pltpu

_E = 64
_T = 2048
_H = 768
_F = 3072
_CAP = _T // _E


def _body(x_ref, rw_ref, w1_ref, w2_ref, y_ref, s_ref, wt_ref):
    i = pl.program_id(0)

    @pl.when(i == 0)
    def _routing():
        xf = x_ref[...]
        logits = jnp.dot(xf, rw_ref[...], preferred_element_type=jnp.float32)
        m = jnp.max(logits, axis=1, keepdims=True)
        denom = jnp.sum(jnp.exp(logits - m), axis=1, keepdims=True)
        wt = 1.0 / denom
        iota_e = jax.lax.broadcasted_iota(jnp.int32, (_T, _E), 1).astype(
            jnp.float32)
        eid = jnp.min(jnp.where(logits == m, iota_e, jnp.float32(_E)),
                      axis=1, keepdims=True)
        onehot = (iota_e == eid).astype(jnp.float32)
        c = onehot
        k = 1
        while k < _T:
            c = c + jnp.concatenate(
                [jnp.zeros((k, _E), jnp.float32), c[:-k]], axis=0)
            k *= 2
        rank = jnp.sum(c * onehot, axis=1, keepdims=True) - 1.0
        s_ref[...] = jnp.where(rank < float(_CAP), eid * float(_CAP) + rank,
                               -1.0)
        wt_ref[...] = wt
        y_ref[...] = jnp.zeros_like(y_ref)

    @pl.when(i > 0)
    def _expert():
        slots = (jax.lax.broadcasted_iota(jnp.int32, (1, _CAP), 1)
                 + (i - 1) * _CAP).astype(jnp.float32)
        de = (s_ref[...] == slots).astype(jnp.float32)
        g = jax.lax.dot_general(de, x_ref[...], (((0,), (0,)), ((), ())),
                                preferred_element_type=jnp.float32)
        h = jnp.dot(g, w1_ref[0], preferred_element_type=jnp.float32)
        h = 0.5 * h * (1.0 + jnp.tanh(0.7978845608028654
                                      * (h + 0.044715 * (h * h * h))))
        out = jnp.dot(h, w2_ref[0], preferred_element_type=jnp.float32)
        y_ref[...] += jax.lax.dot_general(
            de * wt_ref[...], out, (((1,), (0,)), ((), ())),
            preferred_element_type=jnp.float32)


def kernel(x, router_w, w1, w2, bias):
    xf = x.reshape(_T, _H)
    y = pl.pallas_call(
        _body,
        grid=(_E + 1,),
        in_specs=[
            pl.BlockSpec((_T, _H), lambda i: (0, 0)),
            pl.BlockSpec((_H, _E), lambda i: (0, 0)),
            pl.BlockSpec((1, _H, _F), lambda i: (jnp.maximum(i - 1, 0), 0, 0)),
            pl.BlockSpec((1, _F, _H), lambda i: (jnp.maximum(i - 1, 0), 0, 0)),
        ],
        out_specs=pl.BlockSpec((_T, _H), lambda i: (0, 0)),
        out_shape=jax.ShapeDtypeStruct((_T, _H), jnp.float32),
        scratch_shapes=[pltpu.VMEM((_T, 1), jnp.float32),
                        pltpu.VMEM((_T, 1), jnp.float32)],
    )(xf, router_w, w1, w2)
    return (y.reshape(x.shape), bias)
